# baseline (speedup 1.0000x reference)
import jax
import jax.numpy as jnp
from jax.experimental import pallas as pl
from jax.experimental.pallas import tpu as pltpu

B = 1024
C = 3
HW = 224 * 224
E = 64
BC = 1024
CB_PER_CH = HW // BC
NBLK = C * CB_PER_CH


def _router_kernel(x_ref, wt_ref, b_ref, gates_ref, idx_ref, acc_ref):
    j = pl.program_id(0)

    @pl.when(j == 0)
    def _init():
        acc_ref[...] = jnp.zeros_like(acc_ref)

    c = j // CB_PER_CH
    xb = x_ref[...]
    partial = xb[:, 0:128]
    for k in range(1, BC // 128):
        partial = partial + xb[:, k * 128:(k + 1) * 128]
    acc_ref[c] = acc_ref[c] + partial

    @pl.when(j == NBLK - 1)
    def _finish():
        n = jnp.float32(HW)
        p0 = (jnp.sum(acc_ref[0], axis=1, keepdims=True) / n).astype(
            jnp.bfloat16).astype(jnp.float32)
        p1 = (jnp.sum(acc_ref[1], axis=1, keepdims=True) / n).astype(
            jnp.bfloat16).astype(jnp.float32)
        p2 = (jnp.sum(acc_ref[2], axis=1, keepdims=True) / n).astype(
            jnp.bfloat16).astype(jnp.float32)
        wb = wt_ref[...].astype(jnp.bfloat16).astype(jnp.float32)
        logits = (p0 * wb[0:1, :] + p1 * wb[1:2, :]) + p2 * wb[2:3, :]
        logits = logits + b_ref[...]

        iota = jax.lax.broadcasted_iota(jnp.int32, (B, E), 1)
        m0 = jnp.max(logits, axis=1, keepdims=True)
        idx0 = jnp.min(jnp.where(logits == m0, iota, E), axis=1, keepdims=True)
        masked = jnp.where(iota == idx0, jnp.finfo(jnp.float32).min, logits)
        m1 = jnp.max(masked, axis=1, keepdims=True)
        idx1 = jnp.min(jnp.where(masked == m1, iota, E), axis=1, keepdims=True)

        e1 = jnp.exp(m1 - m0)
        denom = 1.0 + e1
        g0 = 1.0 / denom
        g1 = e1 / denom

        gates_ref[...] = jnp.where(iota == idx0, g0,
                                   jnp.where(iota == idx1, g1, 0.0))
        idx_ref[...] = jnp.concatenate([idx0, idx1], axis=1)


def _build():
    return pl.pallas_call(
        _router_kernel,
        grid=(NBLK,),
        in_specs=[
            pl.BlockSpec((B, BC), lambda j: (0, j)),
            pl.BlockSpec((C, E), lambda j: (0, 0)),
            pl.BlockSpec((1, E), lambda j: (0, 0)),
        ],
        out_specs=[
            pl.BlockSpec((B, E), lambda j: (0, 0)),
            pl.BlockSpec((B, 2), lambda j: (0, 0)),
        ],
        out_shape=[
            jax.ShapeDtypeStruct((B, E), jnp.float32),
            jax.ShapeDtypeStruct((B, 2), jnp.int32),
        ],
        scratch_shapes=[pltpu.VMEM((C, B, 128), jnp.float32)],
    )


def kernel(x, W, b):
    xf = x.reshape(B, C * HW)
    wt = W.T
    b2 = b.reshape(1, E)
    gates, idx = _build()(xf, wt, b2)
    return (gates, idx)
